# Initial kernel scaffold; baseline (speedup 1.0000x reference)
import jax
import jax.numpy as jnp
from jax.experimental import pallas as pl


def kernel(features, idx):
    raise NotImplementedError("write your pallas kernel here")


# speedup vs baseline: 1.8613x; 1.8613x over previous
import functools

import jax
import jax.numpy as jnp
from jax import lax
from jax.experimental import pallas as pl
from jax.experimental.pallas import tpu as pltpu
from jax.experimental.pallas import tpu_sc as plsc

_LANES = 16


def _build_sc_gather(num_rows, n, m, rows_per_batch):
    info = plsc.get_sparse_core_info()
    num_workers = info.num_cores * info.num_subcores
    assert num_rows % num_workers == 0
    rows_per_w = num_rows // num_workers
    assert rows_per_batch % rows_per_w == 0
    assert m % _LANES == 0

    def body(feat_hbm, idx_hbm, out_hbm, idx_v, row_v, out_v):
        w = lax.axis_index("s") * info.num_cores + lax.axis_index("c")
        row0 = w * rows_per_w
        b = row0 // rows_per_batch
        pltpu.sync_copy(idx_hbm.at[b], idx_v)

        def per_row(j, carry):
            r = row0 + j
            pltpu.sync_copy(feat_hbm.at[r], row_v)

            def gath(i, carry2):
                s = i * _LANES
                iv = idx_v[pl.ds(s, _LANES)]
                out_v[pl.ds(s, _LANES)] = plsc.load_gather(row_v, [iv])
                return carry2

            lax.fori_loop(0, m // _LANES, gath, 0)
            pltpu.sync_copy(out_v, out_hbm.at[r])
            return carry

        lax.fori_loop(0, rows_per_w, per_row, 0)

    return pl.kernel(
        body,
        out_type=jax.ShapeDtypeStruct((num_rows, m), jnp.float32),
        mesh=plsc.VectorSubcoreMesh(core_axis_name="c", subcore_axis_name="s"),
        scratch_types=[
            pltpu.VMEM((m,), jnp.int32),
            pltpu.VMEM((n,), jnp.float32),
            pltpu.VMEM((m,), jnp.float32),
        ],
        compiler_params=pltpu.CompilerParams(needs_layout_passes=False),
    )


def kernel(features, idx):
    b, c, n = features.shape
    m = idx.shape[1]
    feat2d = features.reshape(b * c, n)
    idx32 = idx.astype(jnp.int32)
    gather = _build_sc_gather(b * c, n, m, c)
    out = gather(feat2d, idx32)
    return out.reshape(b, c, m)


# speedup vs baseline: 3.5209x; 1.8917x over previous
import jax
import jax.numpy as jnp
from jax import lax
from jax.experimental import pallas as pl
from jax.experimental.pallas import tpu as pltpu
from jax.experimental.pallas import tpu_sc as plsc

_LANES = 16
_CHUNK = 4096
_UNROLL = 8


def _build_sc_gather(num_rows, n, m, rows_per_batch):
    info = plsc.get_sparse_core_info()
    num_workers = info.num_cores * info.num_subcores
    assert num_rows % num_workers == 0
    rows_per_w = num_rows // num_workers
    assert rows_per_batch % rows_per_w == 0
    assert rows_per_w % 2 == 0
    assert m % _CHUNK == 0
    nch = m // _CHUNK

    def body(feat_hbm, idx_hbm, out_hbm, idx_v, row_a, row_b, out_v,
             in0, in1, o0, o1):
        rows = (row_a, row_b)
        insems = (in0, in1)
        osems = (o0, o1)
        w = lax.axis_index("s") * info.num_cores + lax.axis_index("c")
        row0 = w * rows_per_w
        b = row0 // rows_per_batch
        pltpu.sync_copy(idx_hbm.at[b], idx_v)
        pltpu.async_copy(feat_hbm.at[row0], rows[0], insems[0])
        pltpu.async_copy(feat_hbm.at[row0 + 1], rows[1], insems[1])

        def out_wait(p):
            pltpu.make_async_copy(
                out_v.at[p], out_hbm.at[0, pl.ds(0, _CHUNK)], osems[p]
            ).wait()

        def per_group(g, carry):
            for k in range(2):
                j = g * 2 + k
                r = row0 + j
                pltpu.make_async_copy(
                    feat_hbm.at[r], rows[k], insems[k]
                ).wait()
                for cc in range(nch):
                    p = cc % 2
                    if cc >= 2:
                        out_wait(p)
                    else:
                        @pl.when(j > 0)
                        def _():
                            out_wait(p)
                    base = cc * _CHUNK

                    @plsc.parallel_loop(0, _CHUNK // _LANES, unroll=_UNROLL)
                    def _(i):
                        iv = idx_v[pl.ds(base + i * _LANES, _LANES)]
                        out_v[p, pl.ds(i * _LANES, _LANES)] = plsc.load_gather(
                            rows[k], [iv]
                        )

                    pltpu.async_copy(
                        out_v.at[p], out_hbm.at[r, pl.ds(base, _CHUNK)], osems[p]
                    )

                @pl.when(j + 2 < rows_per_w)
                def _():
                    pltpu.async_copy(feat_hbm.at[r + 2], rows[k], insems[k])

            return carry

        lax.fori_loop(0, rows_per_w // 2, per_group, 0)
        out_wait(0)
        out_wait(1)

    return pl.kernel(
        body,
        out_type=jax.ShapeDtypeStruct((num_rows, m), jnp.float32),
        mesh=plsc.VectorSubcoreMesh(core_axis_name="c", subcore_axis_name="s"),
        scratch_types=[
            pltpu.VMEM((m,), jnp.int32),
            pltpu.VMEM((n,), jnp.float32),
            pltpu.VMEM((n,), jnp.float32),
            pltpu.VMEM((2, _CHUNK), jnp.float32),
            pltpu.SemaphoreType.DMA,
            pltpu.SemaphoreType.DMA,
            pltpu.SemaphoreType.DMA,
            pltpu.SemaphoreType.DMA,
        ],
        compiler_params=pltpu.CompilerParams(needs_layout_passes=False),
    )


def kernel(features, idx):
    b, c, n = features.shape
    m = idx.shape[1]
    feat2d = features.reshape(b * c, n)
    idx32 = idx.astype(jnp.int32)
    gather = _build_sc_gather(b * c, n, m, c)
    out = gather(feat2d, idx32)
    return out.reshape(b, c, m)
